# baseline (speedup 1.0000x reference)
import functools

import jax
import jax.numpy as jnp
from jax import lax
from jax.experimental import pallas as pl
from jax.experimental.pallas import tpu as pltpu
from jax.experimental.pallas import tpu_sc as plsc

_N_USERS = 50000
_N_SHARED = 1024
_D = 64
_TEMP = 5.0
_NODES = 75000
_NPAD = 75008
_E = 800000
_EP = 819200
_IDXW = 128
_ROWS = _EP // _IDXW
_NSC = 16
_WROWS = _ROWS // _NSC
_CH = 8
_CE = _CH * _IDXW
_NCHUNK = _WROWS // _CH
_ZR = 512


@functools.lru_cache(maxsize=None)
def _make_spmm():
    f32 = jnp.float32
    i32 = jnp.int32
    mesh = plsc.VectorSubcoreMesh(core_axis_name="c", subcore_axis_name="s",
                                  num_cores=2, num_subcores=_NSC)
    out_t = [jax.ShapeDtypeStruct((_NPAD, 16), f32) for _ in range(8)]
    scratch = [
        pltpu.VMEM((3, 3, _CE), i32),
        pltpu.VMEM((2 * _CE, 16), f32),
        pltpu.VMEM((_ZR, 16), f32),
        pltpu.VMEM_SHARED((_NPAD, 16), f32),
        pltpu.SemaphoreType.DMA,
        pltpu.SemaphoreType.DMA,
        pltpu.SemaphoreType.DMA,
    ]

    @functools.partial(
        pl.kernel, out_type=out_t, mesh=mesh, scratch_types=scratch,
        compiler_params=pltpu.CompilerParams(use_tc_tiling_on_sc=False,
                                             needs_layout_passes=False))
    def spmm(edges_s, edges_t, xs0, xs1, xs2, xs3, xt0, xt1, xt2, xt3,
             os0, os1, os2, os3, ot0, ot1, ot2, ot3,
             evb, rows, zbuf, acc, gsem, ssem, esem):
        c = lax.axis_index("c")
        s = lax.axis_index("s")

        def zb(i, carry):
            zbuf[i, :] = jnp.zeros((16,), f32)
            return carry
        lax.fori_loop(0, _ZR, zb, 0)

        xs = ((xs0, xs1, xs2, xs3), (xt0, xt1, xt2, xt3))
        outs = ((os0, os1, os2, os3), (ot0, ot1, ot2, ot3))
        eds = (edges_s, edges_t)
        nzr = _NPAD // _NSC
        r0 = s * nzr
        n_full = nzr // _ZR
        rem = nzr - n_full * _ZR

        def start_load_edges(edges, k, eslot):
            pltpu.async_copy(
                edges.at[s * _NCHUNK + k], evb.at[eslot], esem)

        def wait_load_edges(edges, k, eslot):
            pltpu.make_async_copy(
                edges.at[s * _NCHUNK + k], evb.at[eslot], esem).wait()

        def fire_gathers(xslab, eslot, rslot):
            pltpu.async_copy(
                xslab.at[evb.at[eslot, 0]],
                rows.at[pl.ds(rslot * _CE, _CE)], gsem)

        def drain_gathers(xslab, eslot, rslot):
            pltpu.make_async_copy(
                xslab.at[evb.at[eslot, 0]],
                rows.at[pl.ds(rslot * _CE, _CE)], gsem).wait()

        def scatter(eslot, rslot):
            pltpu.async_copy(
                rows.at[pl.ds(rslot * _CE, _CE)],
                acc.at[evb.at[eslot, 1]], ssem, add=True)
            pltpu.make_async_copy(
                rows.at[pl.ds(rslot * _CE, _CE)],
                acc.at[evb.at[eslot, 1]], ssem).wait()

        def scale(eslot, rslot):
            def body(g, carry):
                vv = plsc.bitcast(evb[eslot, 2, pl.ds(g * 16, 16)], f32)
                for l in range(16):
                    r = rslot * _CE + g * 16 + l
                    bl = vv.at[jnp.full((16,), l, i32)].get(
                        mode="promise_in_bounds")
                    rows[r, :] = rows[r, :] * bl
                return carry
            lax.fori_loop(0, _CE // 16, body, 0)

        def run_slice(m, j):
            edges = eds[m]
            xslab = xs[m][j]
            oslab = outs[m][j]
            for t in range(n_full):
                pltpu.sync_copy(zbuf, acc.at[pl.ds(r0 + t * _ZR, _ZR)])
            if rem:
                pltpu.sync_copy(zbuf.at[pl.ds(0, rem)],
                                acc.at[pl.ds(r0 + n_full * _ZR, rem)])
            plsc.subcore_barrier()

            start_load_edges(edges, 0, 0)
            wait_load_edges(edges, 0, 0)
            fire_gathers(xslab, 0, 0)
            start_load_edges(edges, 1, 1)

            def chunk(k, carry):
                rslot = lax.rem(k, 2)
                eslot = lax.rem(k, 3)
                nrslot = 1 - rslot
                neslot = lax.rem(k + 1, 3)

                @pl.when(k < _NCHUNK - 1)
                def _():
                    wait_load_edges(edges, k + 1, neslot)
                    fire_gathers(xslab, neslot, nrslot)

                    @pl.when(k < _NCHUNK - 2)
                    def _():
                        start_load_edges(edges, k + 2, lax.rem(k + 2, 3))
                drain_gathers(xslab, eslot, rslot)
                scale(eslot, rslot)
                scatter(eslot, rslot)
                return carry
            lax.fori_loop(0, _NCHUNK, chunk, 0)
            plsc.subcore_barrier()
            pltpu.sync_copy(acc.at[pl.ds(r0, nzr)],
                            oslab.at[pl.ds(r0, nzr)])

        for m in range(2):
            for j in range(4):
                @pl.when(j // 2 == c)
                def _(m=m, j=j):
                    run_slice(m, j)

    return spmm


def _spmm(*args):
    return _make_spmm()(*args)


def _inter_body(src_ref, tgt_ref, map_ref, src3_ref, tgt3_ref):
    f32 = jnp.float32
    src = src_ref[...]
    tgt = tgt_ref[...]
    mp = map_ref[...]
    dn = (((1,), (1,)), ((), ()))
    d0 = (((0,), (0,)), ((), ()))
    a = jnp.dot(src, mp, preferred_element_type=f32)
    s = jnp.exp(lax.dot_general(a, tgt, dn, preferred_element_type=f32)
                / _TEMP)
    sr = s / jnp.maximum(jnp.sum(s, axis=1, keepdims=True), 1e-12)
    sc_ = s / jnp.maximum(jnp.sum(s, axis=0, keepdims=True), 1e-12)
    src2 = src + jnp.dot(sr, tgt, preferred_element_type=f32)
    tgt2 = tgt + lax.dot_general(sc_, src2, d0, preferred_element_type=f32)
    ri = lax.broadcasted_iota(jnp.int32, (_N_SHARED, _N_SHARED), 0)
    ci = lax.broadcasted_iota(jnp.int32, (_N_SHARED, _N_SHARED), 1)
    eye = (ri == ci).astype(f32)
    ssT = lax.dot_general(s, s, dn, preferred_element_type=f32) + eye
    sTs = lax.dot_general(s, s, d0, preferred_element_type=f32) + eye
    adj_s = ssT / jnp.sum(ssT, axis=1, keepdims=True)
    adj_t = sTs / jnp.sum(sTs, axis=1, keepdims=True)
    src3_ref[...] = jnp.dot(adj_s, src2, preferred_element_type=f32)
    tgt3_ref[...] = jnp.dot(adj_t, tgt2, preferred_element_type=f32)


def _inter(src, tgt, mapping):
    return pl.pallas_call(
        _inter_body,
        out_shape=(jax.ShapeDtypeStruct((_N_SHARED, _D), jnp.float32),
                   jax.ShapeDtypeStruct((_N_SHARED, _D), jnp.float32)),
    )(src, tgt, mapping)


def _prep_edges(idx, val):
    pad = _EP - _E
    src = jnp.concatenate([idx[1], jnp.zeros((pad,), idx.dtype)])
    dst = jnp.concatenate([idx[0], jnp.zeros((pad,), idx.dtype)])
    v = jnp.concatenate([val, jnp.zeros((pad,), val.dtype)])
    return jnp.stack(
        [src.astype(jnp.int32).reshape(_EP // _CE, _CE),
         dst.astype(jnp.int32).reshape(_EP // _CE, _CE),
         jax.lax.bitcast_convert_type(v, jnp.int32).reshape(_EP // _CE, _CE)],
        axis=1)


def _to_slabs(x):
    xp = jnp.concatenate([x, jnp.zeros((_NPAD - _NODES, _D), x.dtype)])
    return [xp[:, j * 16:(j + 1) * 16] for j in range(4)]


def _dense(slabs, n):
    return jnp.concatenate([sl[:n] for sl in slabs], axis=1)


def kernel(adj_s_idx, adj_s_val, adj_t_idx, adj_t_val, su, tu, si, ti,
           mapping):
    es = _prep_edges(adj_s_idx, adj_s_val)
    et = _prep_edges(adj_t_idx, adj_t_val)
    slabs_s = _to_slabs(jnp.concatenate([su, si], axis=0))
    slabs_t = _to_slabs(jnp.concatenate([tu, ti], axis=0))

    src_slabs = [slabs_s]
    tgt_slabs = [slabs_t]
    for _ in range(3):
        res = _spmm(es, et, *slabs_s, *slabs_t)
        slabs_s, slabs_t = list(res[:4]), list(res[4:])
        head_s = _dense(slabs_s, _N_SHARED)
        head_t = _dense(slabs_t, _N_SHARED)
        src3, tgt3 = _inter(head_s, head_t, mapping)
        slabs_s = [sl.at[:_N_SHARED].set(src3[:, j * 16:(j + 1) * 16])
                   for j, sl in enumerate(slabs_s)]
        slabs_t = [sl.at[:_N_SHARED].set(tgt3[:, j * 16:(j + 1) * 16])
                   for j, sl in enumerate(slabs_t)]
        src_slabs.append(slabs_s)
        tgt_slabs.append(slabs_t)

    source_final = jnp.concatenate(
        [sl[:_NODES] for layer in src_slabs for sl in layer], axis=1)
    target_final = jnp.concatenate(
        [sl[:_NODES] for layer in tgt_slabs for sl in layer], axis=1)
    return (source_final[:_N_USERS], source_final[_N_USERS:],
            target_final[:_N_USERS], target_final[_N_USERS:])


# speedup vs baseline: 1.0532x; 1.0532x over previous
import functools

import jax
import jax.numpy as jnp
from jax import lax
from jax.experimental import pallas as pl
from jax.experimental.pallas import tpu as pltpu
from jax.experimental.pallas import tpu_sc as plsc

_N_USERS = 50000
_N_SHARED = 1024
_D = 64
_TEMP = 5.0
_NODES = 75000
_NPAD = 75008
_E = 800000
_EP = 819200
_IDXW = 128
_ROWS = _EP // _IDXW
_NSC = 16
_WROWS = _ROWS // _NSC
_CH = 8
_CE = _CH * _IDXW
_NCHUNK = _WROWS // _CH
_ZR = 512


@functools.lru_cache(maxsize=None)
def _make_spmm():
    f32 = jnp.float32
    i32 = jnp.int32
    mesh = plsc.VectorSubcoreMesh(core_axis_name="c", subcore_axis_name="s",
                                  num_cores=2, num_subcores=_NSC)
    out_t = [jax.ShapeDtypeStruct((_NPAD, 16), f32) for _ in range(4)]
    scratch = [
        pltpu.VMEM((3, 3, _CE), i32),
        pltpu.VMEM((2 * _CE, 16), f32),
        pltpu.VMEM((_ZR, 16), f32),
        pltpu.VMEM_SHARED((_NPAD, 16), f32),
        pltpu.SemaphoreType.DMA,
        pltpu.SemaphoreType.DMA,
        pltpu.SemaphoreType.DMA,
    ]

    @functools.partial(
        pl.kernel, out_type=out_t, mesh=mesh, scratch_types=scratch,
        compiler_params=pltpu.CompilerParams(use_tc_tiling_on_sc=False,
                                             needs_layout_passes=False))
    def spmm(edges, x0, x1, x2, x3, o0, o1, o2, o3,
             evb, rows, zbuf, acc, gsem, ssem, esem):
        c = lax.axis_index("c")
        s = lax.axis_index("s")

        def zb(i, carry):
            zbuf[i, :] = jnp.zeros((16,), f32)
            return carry
        lax.fori_loop(0, _ZR, zb, 0)

        xs = (x0, x1, x2, x3)
        outs = (o0, o1, o2, o3)
        nzr = _NPAD // _NSC
        r0 = s * nzr
        n_full = nzr // _ZR
        rem = nzr - n_full * _ZR

        def start_load_edges(edges, k, eslot):
            pltpu.async_copy(
                edges.at[s * _NCHUNK + k], evb.at[eslot], esem)

        def wait_load_edges(edges, k, eslot):
            pltpu.make_async_copy(
                edges.at[s * _NCHUNK + k], evb.at[eslot], esem).wait()

        def fire_gathers(xslab, eslot, rslot):
            pltpu.async_copy(
                xslab.at[evb.at[eslot, 0]],
                rows.at[pl.ds(rslot * _CE, _CE)], gsem)

        def drain_gathers(xslab, eslot, rslot):
            pltpu.make_async_copy(
                xslab.at[evb.at[eslot, 0]],
                rows.at[pl.ds(rslot * _CE, _CE)], gsem).wait()

        def scatter(eslot, rslot):
            pltpu.async_copy(
                rows.at[pl.ds(rslot * _CE, _CE)],
                acc.at[evb.at[eslot, 1]], ssem, add=True)
            pltpu.make_async_copy(
                rows.at[pl.ds(rslot * _CE, _CE)],
                acc.at[evb.at[eslot, 1]], ssem).wait()

        def scale(eslot, rslot):
            def body(g, carry):
                vv = plsc.bitcast(evb[eslot, 2, pl.ds(g * 16, 16)], f32)
                for l in range(16):
                    r = rslot * _CE + g * 16 + l
                    bl = vv.at[jnp.full((16,), l, i32)].get(
                        mode="promise_in_bounds")
                    rows[r, :] = rows[r, :] * bl
                return carry
            lax.fori_loop(0, _CE // 16, body, 0)

        def run_slice(j):
            xslab = xs[j]
            oslab = outs[j]
            for t in range(n_full):
                pltpu.sync_copy(zbuf, acc.at[pl.ds(r0 + t * _ZR, _ZR)])
            if rem:
                pltpu.sync_copy(zbuf.at[pl.ds(0, rem)],
                                acc.at[pl.ds(r0 + n_full * _ZR, rem)])
            plsc.subcore_barrier()

            start_load_edges(edges, 0, 0)
            wait_load_edges(edges, 0, 0)
            fire_gathers(xslab, 0, 0)
            start_load_edges(edges, 1, 1)

            def chunk(k, carry):
                rslot = lax.rem(k, 2)
                eslot = lax.rem(k, 3)
                nrslot = 1 - rslot
                neslot = lax.rem(k + 1, 3)

                @pl.when(k < _NCHUNK - 1)
                def _():
                    wait_load_edges(edges, k + 1, neslot)
                    fire_gathers(xslab, neslot, nrslot)

                    @pl.when(k < _NCHUNK - 2)
                    def _():
                        start_load_edges(edges, k + 2, lax.rem(k + 2, 3))
                drain_gathers(xslab, eslot, rslot)
                scale(eslot, rslot)
                scatter(eslot, rslot)
                return carry
            lax.fori_loop(0, _NCHUNK, chunk, 0)
            plsc.subcore_barrier()
            pltpu.sync_copy(acc.at[pl.ds(r0, nzr)],
                            oslab.at[pl.ds(r0, nzr)])

        for j in range(4):
            @pl.when(j // 2 == c)
            def _(j=j):
                run_slice(j)

    return spmm


def _spmm(*args):
    return _make_spmm()(*args)


def _inter_body(src_ref, tgt_ref, map_ref, src3_ref, tgt3_ref):
    f32 = jnp.float32
    src = src_ref[...]
    tgt = tgt_ref[...]
    mp = map_ref[...]
    dn = (((1,), (1,)), ((), ()))
    d0 = (((0,), (0,)), ((), ()))
    a = jnp.dot(src, mp, preferred_element_type=f32)
    s = jnp.exp(lax.dot_general(a, tgt, dn, preferred_element_type=f32)
                / _TEMP)
    sr = s / jnp.maximum(jnp.sum(s, axis=1, keepdims=True), 1e-12)
    sc_ = s / jnp.maximum(jnp.sum(s, axis=0, keepdims=True), 1e-12)
    src2 = src + jnp.dot(sr, tgt, preferred_element_type=f32)
    tgt2 = tgt + lax.dot_general(sc_, src2, d0, preferred_element_type=f32)
    ri = lax.broadcasted_iota(jnp.int32, (_N_SHARED, _N_SHARED), 0)
    ci = lax.broadcasted_iota(jnp.int32, (_N_SHARED, _N_SHARED), 1)
    eye = (ri == ci).astype(f32)
    ssT = lax.dot_general(s, s, dn, preferred_element_type=f32) + eye
    sTs = lax.dot_general(s, s, d0, preferred_element_type=f32) + eye
    adj_s = ssT / jnp.sum(ssT, axis=1, keepdims=True)
    adj_t = sTs / jnp.sum(sTs, axis=1, keepdims=True)
    src3_ref[...] = jnp.dot(adj_s, src2, preferred_element_type=f32)
    tgt3_ref[...] = jnp.dot(adj_t, tgt2, preferred_element_type=f32)


def _inter(src, tgt, mapping):
    return pl.pallas_call(
        _inter_body,
        out_shape=(jax.ShapeDtypeStruct((_N_SHARED, _D), jnp.float32),
                   jax.ShapeDtypeStruct((_N_SHARED, _D), jnp.float32)),
    )(src, tgt, mapping)


def _prep_edges(idx, val):
    pad = _EP - _E
    src = jnp.concatenate([idx[1], jnp.zeros((pad,), idx.dtype)])
    dst = jnp.concatenate([idx[0], jnp.zeros((pad,), idx.dtype)])
    v = jnp.concatenate([val, jnp.zeros((pad,), val.dtype)])
    return jnp.stack(
        [src.astype(jnp.int32).reshape(_EP // _CE, _CE),
         dst.astype(jnp.int32).reshape(_EP // _CE, _CE),
         jax.lax.bitcast_convert_type(v, jnp.int32).reshape(_EP // _CE, _CE)],
        axis=1)


def _to_slabs(x):
    xp = jnp.concatenate([x, jnp.zeros((_NPAD - _NODES, _D), x.dtype)])
    return [xp[:, j * 16:(j + 1) * 16] for j in range(4)]


def _dense(slabs, n):
    return jnp.concatenate([sl[:n] for sl in slabs], axis=1)


def kernel(adj_s_idx, adj_s_val, adj_t_idx, adj_t_val, su, tu, si, ti,
           mapping):
    es = _prep_edges(adj_s_idx, adj_s_val)
    et = _prep_edges(adj_t_idx, adj_t_val)
    slabs_s = _to_slabs(jnp.concatenate([su, si], axis=0))
    slabs_t = _to_slabs(jnp.concatenate([tu, ti], axis=0))

    src_slabs = [slabs_s]
    tgt_slabs = [slabs_t]
    for _ in range(3):
        slabs_s = list(_spmm(es, *slabs_s))
        slabs_t = list(_spmm(et, *slabs_t))
        head_s = _dense(slabs_s, _N_SHARED)
        head_t = _dense(slabs_t, _N_SHARED)
        src3, tgt3 = _inter(head_s, head_t, mapping)
        slabs_s = [sl.at[:_N_SHARED].set(src3[:, j * 16:(j + 1) * 16])
                   for j, sl in enumerate(slabs_s)]
        slabs_t = [sl.at[:_N_SHARED].set(tgt3[:, j * 16:(j + 1) * 16])
                   for j, sl in enumerate(slabs_t)]
        src_slabs.append(slabs_s)
        tgt_slabs.append(slabs_t)

    source_final = jnp.concatenate(
        [sl[:_NODES] for layer in src_slabs for sl in layer], axis=1)
    target_final = jnp.concatenate(
        [sl[:_NODES] for layer in tgt_slabs for sl in layer], axis=1)
    return (source_final[:_N_USERS], source_final[_N_USERS:],
            target_final[:_N_USERS], target_final[_N_USERS:])
